# baseline (device time: 290523 ns/iter reference)
import math

import jax
import jax.numpy as jnp
from jax import lax
from jax.experimental import pallas as pl
from jax.experimental.pallas import tpu as pltpu

N_DEV = 4
B = 2
SQ = 512
SG = N_DEV * SQ
D = 1024
H = 8
DH = 128
SCALE = 0.08838834764831843
LOG_BASE = math.log(10000.0)


def _body(x_ref, wq_ref, wk_ref, wv_ref, wo_ref, out_ref,
          xg_ref, k_ref, v_ref, qbuf_ref, sbuf_ref, rbuf_ref,
          ag_send, ag_recv, rs_send, rs_recv):
    me = lax.axis_index("i")
    right = (me + 1) % N_DEV
    left = (me + N_DEV - 1) % N_DEV

    barrier = pltpu.get_barrier_semaphore()
    for nbr in (left, right):
        pl.semaphore_signal(barrier, inc=1, device_id=(nbr,),
                            device_id_type=pl.DeviceIdType.MESH)
    pl.semaphore_wait(barrier, 2)

    didx = lax.broadcasted_iota(jnp.int32, (SQ, D), 1)
    dmod = didx % DH
    even = (dmod % 2) == 0
    fexp = ((dmod // 2) * 2).astype(jnp.float32) * (1.0 / DH)
    invf = jnp.exp(fexp * (-LOG_BASE))
    rows = lax.broadcasted_iota(jnp.int32, (SQ, D), 0)

    def rope_tabs(origin):
        theta = (rows + origin * SQ).astype(jnp.float32) * invf
        cosv = jnp.cos(theta).astype(jnp.bfloat16)
        sin_alt = (jnp.sin(theta) * jnp.where(even, -1.0, 1.0)
                   ).astype(jnp.bfloat16)
        return cosv, sin_alt

    def rope(t32, cosv, sin_alt):
        t = t32.astype(jnp.bfloat16)
        swapped = jnp.where(even, jnp.roll(t, -1, axis=1),
                            jnp.roll(t, 1, axis=1))
        return t * cosv + swapped * sin_alt

    def proj_kv(origin, get_x):
        row0 = origin * SQ
        cosv, sin_alt = rope_tabs(origin)
        for b in range(B):
            xb = get_x(b)
            k32 = jnp.dot(xb, wk_ref[...], preferred_element_type=jnp.float32)
            v32 = jnp.dot(xb, wv_ref[...], preferred_element_type=jnp.float32)
            kr = rope(k32, cosv, sin_alt).astype(jnp.bfloat16)
            k_ref[b, :, pl.ds(row0, SQ)] = jnp.transpose(kr)
            v_ref[b, pl.ds(row0, SQ), :] = v32.astype(jnp.bfloat16)

    for h in range(N_DEV - 1):
        rdma = pltpu.make_async_remote_copy(
            src_ref=x_ref if h == 0 else xg_ref.at[h - 1],
            dst_ref=xg_ref.at[h],
            send_sem=ag_send.at[h],
            recv_sem=ag_recv.at[h],
            device_id=(right,),
            device_id_type=pl.DeviceIdType.MESH,
        )
        rdma.start()
        if h == 0:
            proj_kv(me, lambda b: x_ref[b])
        else:
            proj_kv((me + N_DEV - h) % N_DEV,
                    lambda b: xg_ref[h - 1, b])
        rdma.wait()
    proj_kv((me + 1) % N_DEV, lambda b: xg_ref[N_DEV - 2, b])

    ones_sg = jnp.ones((SG, DH), jnp.bfloat16)
    rs_rdmas = []
    for step in range(N_DEV):
        qc = (me + N_DEV - 1 - step) % N_DEV
        row0 = qc * SQ
        cosv, sin_alt = rope_tabs(qc)
        for b in range(B):
            xb = x_ref[b] if step == N_DEV - 1 else xg_ref[step, b]
            q32 = jnp.dot(xb, wq_ref[...],
                          preferred_element_type=jnp.float32) * SCALE
            qbuf_ref[b] = rope(q32, cosv, sin_alt).astype(jnp.bfloat16)

        SUB = 256

        def head_body(h, parts):
            hd = h * DH
            new = []
            for b in range(B):
                kt = k_ref[b, pl.ds(hd, DH), :]
                vv = jnp.concatenate(
                    [v_ref[b, :, pl.ds(hd, DH)], ones_sg], axis=1)
                wo_h = wo_ref[pl.ds(hd, DH), :]
                subs = []
                for t in range(SQ // SUB):
                    qt = qbuf_ref[b, t * SUB:(t + 1) * SUB, pl.ds(hd, DH)]
                    s = jnp.dot(qt, kt, preferred_element_type=jnp.float32)
                    p = jnp.exp(s).astype(jnp.bfloat16)
                    o2 = jnp.dot(p, vv, preferred_element_type=jnp.float32)
                    ctx = o2[:, :DH] / o2[:, DH:]
                    subs.append(jnp.dot(ctx.astype(jnp.bfloat16), wo_h,
                                        preferred_element_type=jnp.float32))
                new.append(parts[b] + jnp.concatenate(subs, axis=0))
            return tuple(new)

        zero = jnp.zeros((SQ, D), jnp.float32)
        parts = lax.fori_loop(0, H, head_body, (zero,) * B)

        if step > 0:
            rs_rdmas[step - 1].wait_recv()
            rs_rdmas[step - 1].wait_send()
        for b in range(B):
            part = parts[b]
            if step > 0:
                part = part + rbuf_ref[step - 1, b].astype(jnp.float32)
            if step < N_DEV - 1:
                sbuf_ref[step % 2, b] = part.astype(jnp.bfloat16)
            else:
                out_ref[b] = part
        if step < N_DEV - 1:
            rdma = pltpu.make_async_remote_copy(
                src_ref=sbuf_ref.at[step % 2],
                dst_ref=rbuf_ref.at[step],
                send_sem=rs_send.at[step],
                recv_sem=rs_recv.at[step],
                device_id=(right,),
                device_id_type=pl.DeviceIdType.MESH,
            )
            rdma.start()
            rs_rdmas.append(rdma)


def kernel(x, Wq, Wk, Wv, Wo):
    bf = jnp.bfloat16
    return pl.pallas_call(
        _body,
        out_shape=jax.ShapeDtypeStruct((B, SQ, D), jnp.float32),
        in_specs=[pl.BlockSpec(memory_space=pltpu.VMEM)] * 5,
        out_specs=pl.BlockSpec(memory_space=pltpu.VMEM),
        scratch_shapes=[
            pltpu.VMEM((N_DEV - 1, B, SQ, D), bf),
            pltpu.VMEM((B, D, SG), bf),
            pltpu.VMEM((B, SG, D), bf),
            pltpu.VMEM((B, SQ, D), bf),
            pltpu.VMEM((2, B, SQ, D), bf),
            pltpu.VMEM((N_DEV - 1, B, SQ, D), bf),
            pltpu.SemaphoreType.DMA((N_DEV - 1,)),
            pltpu.SemaphoreType.DMA((N_DEV - 1,)),
            pltpu.SemaphoreType.DMA((N_DEV - 1,)),
            pltpu.SemaphoreType.DMA((N_DEV - 1,)),
        ],
        compiler_params=pltpu.CompilerParams(
            collective_id=0, vmem_limit_bytes=100 * 1024 * 1024),
    )(x.astype(bf), Wq.astype(bf), Wk.astype(bf), Wv.astype(bf), Wo.astype(bf))


# device time: 264939 ns/iter; 1.0966x vs baseline; 1.0966x over previous
import math

import jax
import jax.numpy as jnp
from jax import lax
from jax.experimental import pallas as pl
from jax.experimental.pallas import tpu as pltpu

N_DEV = 4
B = 2
SQ = 512
SG = N_DEV * SQ
D = 1024
H = 8
DH = 128
SCALE = 0.08838834764831843
LOG_BASE = math.log(10000.0)


def _body(x_ref, wq_ref, wk_ref, wv_ref, wo_ref, out_ref,
          xg_ref, k_ref, v_ref, qbuf_ref, sbuf_ref, rbuf_ref,
          ag_send, ag_recv, rs_send, rs_recv):
    me = lax.axis_index("i")
    right = (me + 1) % N_DEV
    left = (me + N_DEV - 1) % N_DEV

    barrier = pltpu.get_barrier_semaphore()
    for nbr in (left, right):
        pl.semaphore_signal(barrier, inc=1, device_id=(nbr,),
                            device_id_type=pl.DeviceIdType.MESH)
    pl.semaphore_wait(barrier, 2)

    didx = lax.broadcasted_iota(jnp.int32, (SQ, D), 1)
    dmod = didx % DH
    even = (dmod % 2) == 0
    fexp = ((dmod // 2) * 2).astype(jnp.float32) * (1.0 / DH)
    invf = jnp.exp(fexp * (-LOG_BASE))
    rows = lax.broadcasted_iota(jnp.int32, (SQ, D), 0)

    def rope_tabs(origin):
        theta = (rows + origin * SQ).astype(jnp.float32) * invf
        cosv = jnp.cos(theta).astype(jnp.bfloat16)
        sin_alt = (jnp.sin(theta) * jnp.where(even, -1.0, 1.0)
                   ).astype(jnp.bfloat16)
        return cosv, sin_alt

    def rope(t32, cosv, sin_alt):
        t = t32.astype(jnp.bfloat16)
        swapped = jnp.where(even, jnp.roll(t, -1, axis=1),
                            jnp.roll(t, 1, axis=1))
        return t * cosv + swapped * sin_alt

    def proj_kv(origin, get_x):
        row0 = origin * SQ
        cosv, sin_alt = rope_tabs(origin)
        for b in range(B):
            xb = get_x(b)
            k32 = jnp.dot(xb, wk_ref[...], preferred_element_type=jnp.float32)
            v32 = jnp.dot(xb, wv_ref[...], preferred_element_type=jnp.float32)
            kr = rope(k32, cosv, sin_alt).astype(jnp.bfloat16)
            k_ref[b, :, pl.ds(row0, SQ)] = jnp.transpose(kr)
            v_ref[b, pl.ds(row0, SQ), :] = v32.astype(jnp.bfloat16)

    for h in range(N_DEV - 1):
        rdma = pltpu.make_async_remote_copy(
            src_ref=x_ref if h == 0 else xg_ref.at[h - 1],
            dst_ref=xg_ref.at[h],
            send_sem=ag_send.at[h],
            recv_sem=ag_recv.at[h],
            device_id=(right,),
            device_id_type=pl.DeviceIdType.MESH,
        )
        rdma.start()
        if h == 0:
            proj_kv(me, lambda b: x_ref[b])
        else:
            proj_kv((me + N_DEV - h) % N_DEV,
                    lambda b: xg_ref[h - 1, b])
        rdma.wait()
    proj_kv((me + 1) % N_DEV, lambda b: xg_ref[N_DEV - 2, b])

    rs_rdmas = []
    for step in range(N_DEV):
        qc = (me + N_DEV - 1 - step) % N_DEV
        row0 = qc * SQ
        cosv, sin_alt = rope_tabs(qc)
        for b in range(B):
            xb = x_ref[b] if step == N_DEV - 1 else xg_ref[step, b]
            q32 = jnp.dot(xb, wq_ref[...],
                          preferred_element_type=jnp.float32) * SCALE
            qbuf_ref[b] = rope(q32, cosv, sin_alt).astype(jnp.bfloat16)

        SUB = 256

        def head_body(h, parts):
            hd = h * DH
            new = []
            for b in range(B):
                kt = k_ref[b, pl.ds(hd, DH), :]
                vh = v_ref[b, :, pl.ds(hd, DH)]
                wo_h = wo_ref[pl.ds(hd, DH), :]
                subs = []
                for t in range(SQ // SUB):
                    qt = qbuf_ref[b, t * SUB:(t + 1) * SUB, pl.ds(hd, DH)]
                    s = jnp.dot(qt, kt, preferred_element_type=jnp.float32)
                    p = jnp.exp(s)
                    l = jnp.sum(p, axis=-1, keepdims=True)
                    ctx = jnp.dot(p.astype(jnp.bfloat16), vh,
                                  preferred_element_type=jnp.float32) / l
                    subs.append(jnp.dot(ctx.astype(jnp.bfloat16), wo_h,
                                        preferred_element_type=jnp.float32))
                new.append(parts[b] + jnp.concatenate(subs, axis=0))
            return tuple(new)

        zero = jnp.zeros((SQ, D), jnp.float32)
        parts = lax.fori_loop(0, H, head_body, (zero,) * B)

        if step > 0:
            rs_rdmas[step - 1].wait_recv()
            rs_rdmas[step - 1].wait_send()
        for b in range(B):
            part = parts[b]
            if step > 0:
                part = part + rbuf_ref[step - 1, b].astype(jnp.float32)
            if step < N_DEV - 1:
                sbuf_ref[step % 2, b] = part.astype(jnp.bfloat16)
            else:
                out_ref[b] = part
        if step < N_DEV - 1:
            rdma = pltpu.make_async_remote_copy(
                src_ref=sbuf_ref.at[step % 2],
                dst_ref=rbuf_ref.at[step],
                send_sem=rs_send.at[step],
                recv_sem=rs_recv.at[step],
                device_id=(right,),
                device_id_type=pl.DeviceIdType.MESH,
            )
            rdma.start()
            rs_rdmas.append(rdma)


def kernel(x, Wq, Wk, Wv, Wo):
    bf = jnp.bfloat16
    return pl.pallas_call(
        _body,
        out_shape=jax.ShapeDtypeStruct((B, SQ, D), jnp.float32),
        in_specs=[pl.BlockSpec(memory_space=pltpu.VMEM)] * 5,
        out_specs=pl.BlockSpec(memory_space=pltpu.VMEM),
        scratch_shapes=[
            pltpu.VMEM((N_DEV - 1, B, SQ, D), bf),
            pltpu.VMEM((B, D, SG), bf),
            pltpu.VMEM((B, SG, D), bf),
            pltpu.VMEM((B, SQ, D), bf),
            pltpu.VMEM((2, B, SQ, D), bf),
            pltpu.VMEM((N_DEV - 1, B, SQ, D), bf),
            pltpu.SemaphoreType.DMA((N_DEV - 1,)),
            pltpu.SemaphoreType.DMA((N_DEV - 1,)),
            pltpu.SemaphoreType.DMA((N_DEV - 1,)),
            pltpu.SemaphoreType.DMA((N_DEV - 1,)),
        ],
        compiler_params=pltpu.CompilerParams(
            collective_id=0, vmem_limit_bytes=100 * 1024 * 1024),
    )(x.astype(bf), Wq.astype(bf), Wk.astype(bf), Wv.astype(bf), Wo.astype(bf))
